# baseline (speedup 1.0000x reference)
import jax
import jax.numpy as jnp
from jax import lax
from jax.experimental import pallas as pl
from jax.experimental.pallas import tpu as pltpu
from jax.experimental.pallas import tpu_sc as plsc

RES = 1024
H, W = 1080, 1920
NPIX = H * W
NWIN = 6 * RES * RES * 3 // 8
KMAX = NWIN - 1
C = 1024
NCHUNK = NPIX // C
SUB = 128
NSUB2 = 2 * C // SUB
NLANE = 16
NVREG = C // NLANE
NC, NS = 2, 16
NW = NC * NS

_f32 = jnp.float32
_i32 = jnp.int32


def _pix_coords(x, y, z):
    ax, ay, az = jnp.abs(x), jnp.abs(y), jnp.abs(z)
    is_x = (ax >= ay) & (ax >= az)
    is_y = jnp.logical_not(is_x) & (ay >= az)
    pos_x = x >= 0.0
    pos_y = y >= 0.0
    pos_z = z >= 0.0
    face = jnp.where(
        is_x, jnp.where(pos_x, 0, 1),
        jnp.where(is_y, jnp.where(pos_y, 2, 3),
                  jnp.where(pos_z, 4, 5))).astype(_i32)
    ma = jnp.where(is_x, ax, jnp.where(is_y, ay, az)) + 1e-30
    u = jnp.where(is_x, jnp.where(pos_x, -z, z),
                  jnp.where(is_y, x, jnp.where(pos_z, x, -x)))
    v = jnp.where(is_x, -y,
                  jnp.where(is_y, jnp.where(pos_y, z, -z), -y))
    inv = 1.0 / ma
    s = (u * inv) * (RES * 0.5) + (RES * 0.5 - 0.5)
    t = (v * inv) * (RES * 0.5) + (RES * 0.5 - 0.5)
    si = s.astype(_i32)
    ti = t.astype(_i32)
    s0 = si - jnp.where(s < si.astype(_f32), 1, 0).astype(_i32)
    t0 = ti - jnp.where(t < ti.astype(_f32), 1, 0).astype(_i32)
    ws = s - s0.astype(_f32)
    wt = t - t0.astype(_f32)
    s0c = jnp.clip(s0, 0, RES - 1)
    s1c = jnp.minimum(s0c + 1, RES - 1)
    t0c = jnp.clip(t0, 0, RES - 1)
    t1c = jnp.minimum(t0c + 1, RES - 1)
    sstep = s1c - s0c
    i00 = face * (RES * RES) + t0c * RES + s0c
    i10 = face * (RES * RES) + t1c * RES + s0c
    return i00, i10, sstep, ws, wt


def _body(table, rays, maski, out,
          rays_v, mask_v, it0, it1, off0_v, off1_v, ds_v,
          ws_v, wt_v, mf_v, g0, g1, o0, o1, o2, sem):
    wid = lax.axis_index("s") * NC + lax.axis_index("c")
    cnt = (NCHUNK - wid + NW - 1) // NW
    lane = lax.iota(_i32, NLANE)

    def chunk_body(i, _):
        ch = wid + i * NW
        pbase = ch * C
        pltpu.sync_copy(rays.at[pl.ds(pbase * 3, C * 3)], rays_v)
        pltpu.sync_copy(maski.at[pl.ds(pbase, C)], mask_v)

        def pass1(j, _):
            d = pl.ds(j * NLANE, NLANE)
            i3 = lane * 3 + j * (3 * NLANE)
            x = plsc.load_gather(rays_v, [i3])
            y = plsc.load_gather(rays_v, [i3 + 1])
            z = plsc.load_gather(rays_v, [i3 + 2])
            i00, i10, sstep, ws, wt = _pix_coords(x, y, z)
            m = mask_v[d]
            live = m > 0
            zero = jnp.zeros((NLANE,), _i32)
            w0 = i00 * 3
            w1 = i10 * 3
            k0 = jnp.where(live, lax.shift_right_logical(w0, 3), zero)
            k1 = jnp.where(live, lax.shift_right_logical(w1, 3), zero)
            k0b = jnp.minimum(k0 + 1, KMAX)
            k1b = jnp.minimum(k1 + 1, KMAX)
            row2 = (lane + j * NLANE) * 2
            plsc.store_scatter(it0, [row2], k0)
            plsc.store_scatter(it0, [row2 + 1], k0b)
            plsc.store_scatter(it1, [row2], k1)
            plsc.store_scatter(it1, [row2 + 1], k1b)
            off0_v[d] = w0 & 7
            off1_v[d] = w1 & 7
            ds_v[d] = sstep * 3
            ws_v[d] = ws
            wt_v[d] = wt
            mf_v[d] = m.astype(_f32)
            return _

        lax.fori_loop(0, NVREG, pass1, None)

        copies = []
        for idx_v, g in ((it0, g0), (it1, g1)):
            copies.append(pltpu.async_copy(table.at[idx_v], g, sem))
        for cp in copies:
            cp.wait()

        def pass2(j, _):
            d = pl.ds(j * NLANE, NLANE)
            row2 = (lane + j * NLANE) * 2
            ws = ws_v[d]
            wt = wt_v[d]
            mf = mf_v[d]
            off0 = off0_v[d]
            off1 = off1_v[d]
            ds = ds_v[d]
            for c, o_v in ((0, o0), (1, o1), (2, o2)):
                oa = off0 + c
                ob = oa + ds
                c00 = plsc.load_gather(
                    g0, [row2 + lax.shift_right_logical(oa, 3), oa & 7])
                c01 = plsc.load_gather(
                    g0, [row2 + lax.shift_right_logical(ob, 3), ob & 7])
                oa = off1 + c
                ob = oa + ds
                c10 = plsc.load_gather(
                    g1, [row2 + lax.shift_right_logical(oa, 3), oa & 7])
                c11 = plsc.load_gather(
                    g1, [row2 + lax.shift_right_logical(ob, 3), ob & 7])
                top = c00 + ws * (c01 - c00)
                bot = c10 + ws * (c11 - c10)
                val = top + wt * (bot - top)
                val = jnp.minimum(jnp.maximum(val, 0.0), 1.0) * mf
                o_v[d] = val
            return _

        lax.fori_loop(0, NVREG, pass2, None)

        pltpu.sync_copy(o0, out.at[pl.ds(pbase, C)])
        pltpu.sync_copy(o1, out.at[pl.ds(NPIX + pbase, C)])
        pltpu.sync_copy(o2, out.at[pl.ds(2 * NPIX + pbase, C)])
        return _

    lax.fori_loop(0, cnt, chunk_body, None)


@jax.jit
def _sky(table, rays, maski):
    grid_kernel = pl.kernel(
        _body,
        out_type=jax.ShapeDtypeStruct((3 * NPIX,), _f32),
        mesh=plsc.VectorSubcoreMesh(core_axis_name="c", subcore_axis_name="s"),
        compiler_params=pltpu.CompilerParams(
            needs_layout_passes=False, use_tc_tiling_on_sc=False),
        scratch_types=[
            pltpu.VMEM((C * 3,), _f32),
            pltpu.VMEM((C,), _i32),
            pltpu.VMEM((2 * C,), _i32),
            pltpu.VMEM((2 * C,), _i32),
            pltpu.VMEM((C,), _i32),
            pltpu.VMEM((C,), _i32),
            pltpu.VMEM((C,), _i32),
            pltpu.VMEM((C,), _f32),
            pltpu.VMEM((C,), _f32),
            pltpu.VMEM((C,), _f32),
            pltpu.VMEM((2 * C, 8), _f32),
            pltpu.VMEM((2 * C, 8), _f32),
            pltpu.VMEM((C,), _f32),
            pltpu.VMEM((C,), _f32),
            pltpu.VMEM((C,), _f32),
            pltpu.SemaphoreType.DMA,
        ],
    )
    return grid_kernel(table, rays, maski)


def kernel(sky_cube_map, rays_d, mask):
    table = sky_cube_map.reshape(NWIN, 8)
    rays = rays_d.reshape(NPIX * 3)
    maski = mask.reshape(NPIX).astype(_i32)
    out = _sky(table, rays, maski)
    return out.reshape(3, H, W)


# speedup vs baseline: 1.8921x; 1.8921x over previous
import jax
import jax.numpy as jnp
from jax import lax
from jax.experimental import pallas as pl
from jax.experimental.pallas import tpu as pltpu
from jax.experimental.pallas import tpu_sc as plsc

RES = 1024
H, W = 1080, 1920
NPIX = H * W
NWIN = 6 * RES * RES * 3 // 8
KMAX = NWIN - 1
C = 1024
NCHUNK = NPIX // C
SUB = 128
NSUB2 = 2 * C // SUB
NLANE = 16
NVREG = C // NLANE
NC, NS = 2, 16
NW = NC * NS

_f32 = jnp.float32
_i32 = jnp.int32


def _pix_coords(x, y, z):
    ax, ay, az = jnp.abs(x), jnp.abs(y), jnp.abs(z)
    is_x = (ax >= ay) & (ax >= az)
    is_y = jnp.logical_not(is_x) & (ay >= az)
    pos_x = x >= 0.0
    pos_y = y >= 0.0
    pos_z = z >= 0.0
    face = jnp.where(
        is_x, jnp.where(pos_x, 0, 1),
        jnp.where(is_y, jnp.where(pos_y, 2, 3),
                  jnp.where(pos_z, 4, 5))).astype(_i32)
    ma = jnp.where(is_x, ax, jnp.where(is_y, ay, az)) + 1e-30
    u = jnp.where(is_x, jnp.where(pos_x, -z, z),
                  jnp.where(is_y, x, jnp.where(pos_z, x, -x)))
    v = jnp.where(is_x, -y,
                  jnp.where(is_y, jnp.where(pos_y, z, -z), -y))
    inv = 1.0 / ma
    s = (u * inv) * (RES * 0.5) + (RES * 0.5 - 0.5)
    t = (v * inv) * (RES * 0.5) + (RES * 0.5 - 0.5)
    si = s.astype(_i32)
    ti = t.astype(_i32)
    s0 = si - jnp.where(s < si.astype(_f32), 1, 0).astype(_i32)
    t0 = ti - jnp.where(t < ti.astype(_f32), 1, 0).astype(_i32)
    ws = s - s0.astype(_f32)
    wt = t - t0.astype(_f32)
    s0c = jnp.clip(s0, 0, RES - 1)
    s1c = jnp.minimum(s0c + 1, RES - 1)
    t0c = jnp.clip(t0, 0, RES - 1)
    t1c = jnp.minimum(t0c + 1, RES - 1)
    sstep = s1c - s0c
    i00 = face * (RES * RES) + t0c * RES + s0c
    i10 = face * (RES * RES) + t1c * RES + s0c
    return i00, i10, sstep, ws, wt


def _body(table, rays, maski, out,
          rays_v, mask_v, it0, it1, off0_v, off1_v, ds_v,
          ws_v, wt_v, mf_v, g0, g1, o0, o1, o2, sem):
    wid = lax.axis_index("s") * NC + lax.axis_index("c")
    cnt = (NCHUNK - wid + NW - 1) // NW
    lane = lax.iota(_i32, NLANE)

    def chunk_body(i, _):
        ch = wid + i * NW
        pbase = ch * C
        pltpu.sync_copy(rays.at[pl.ds(pbase * 3, C * 3)], rays_v)
        pltpu.sync_copy(maski.at[pl.ds(pbase, C)], mask_v)

        def pass1(j, _):
            d = pl.ds(j * NLANE, NLANE)
            i3 = lane * 3 + j * (3 * NLANE)
            x = plsc.load_gather(rays_v, [i3])
            y = plsc.load_gather(rays_v, [i3 + 1])
            z = plsc.load_gather(rays_v, [i3 + 2])
            i00, i10, sstep, ws, wt = _pix_coords(x, y, z)
            m = mask_v[d]
            w0 = i00 * 3
            w1 = i10 * 3
            k0 = lax.shift_right_logical(w0, 3)
            k1 = lax.shift_right_logical(w1, 3)
            k0b = jnp.minimum(k0 + 1, KMAX)
            k1b = jnp.minimum(k1 + 1, KMAX)
            row2 = (lane + j * NLANE) * 2
            plsc.store_scatter(it0, [row2], k0)
            plsc.store_scatter(it0, [row2 + 1], k0b)
            plsc.store_scatter(it1, [row2], k1)
            plsc.store_scatter(it1, [row2 + 1], k1b)
            off0_v[d] = w0 & 7
            off1_v[d] = w1 & 7
            ds_v[d] = sstep * 3
            ws_v[d] = ws
            wt_v[d] = wt
            mf_v[d] = m.astype(_f32)
            return _

        lax.fori_loop(0, NVREG, pass1, None)

        copies = []
        for idx_v, g in ((it0, g0), (it1, g1)):
            copies.append(pltpu.async_copy(table.at[idx_v], g, sem))
        for cp in copies:
            cp.wait()

        def pass2(j, _):
            d = pl.ds(j * NLANE, NLANE)
            row2 = (lane + j * NLANE) * 2
            ws = ws_v[d]
            wt = wt_v[d]
            mf = mf_v[d]
            off0 = off0_v[d]
            off1 = off1_v[d]
            ds = ds_v[d]
            for c, o_v in ((0, o0), (1, o1), (2, o2)):
                oa = off0 + c
                ob = oa + ds
                c00 = plsc.load_gather(
                    g0, [row2 + lax.shift_right_logical(oa, 3), oa & 7])
                c01 = plsc.load_gather(
                    g0, [row2 + lax.shift_right_logical(ob, 3), ob & 7])
                oa = off1 + c
                ob = oa + ds
                c10 = plsc.load_gather(
                    g1, [row2 + lax.shift_right_logical(oa, 3), oa & 7])
                c11 = plsc.load_gather(
                    g1, [row2 + lax.shift_right_logical(ob, 3), ob & 7])
                top = c00 + ws * (c01 - c00)
                bot = c10 + ws * (c11 - c10)
                val = top + wt * (bot - top)
                val = jnp.minimum(jnp.maximum(val, 0.0), 1.0) * mf
                o_v[d] = val
            return _

        lax.fori_loop(0, NVREG, pass2, None)

        pltpu.sync_copy(o0, out.at[pl.ds(pbase, C)])
        pltpu.sync_copy(o1, out.at[pl.ds(NPIX + pbase, C)])
        pltpu.sync_copy(o2, out.at[pl.ds(2 * NPIX + pbase, C)])
        return _

    lax.fori_loop(0, cnt, chunk_body, None)


@jax.jit
def _sky(table, rays, maski):
    grid_kernel = pl.kernel(
        _body,
        out_type=jax.ShapeDtypeStruct((3 * NPIX,), _f32),
        mesh=plsc.VectorSubcoreMesh(core_axis_name="c", subcore_axis_name="s"),
        compiler_params=pltpu.CompilerParams(
            needs_layout_passes=False, use_tc_tiling_on_sc=False),
        scratch_types=[
            pltpu.VMEM((C * 3,), _f32),
            pltpu.VMEM((C,), _i32),
            pltpu.VMEM((2 * C,), _i32),
            pltpu.VMEM((2 * C,), _i32),
            pltpu.VMEM((C,), _i32),
            pltpu.VMEM((C,), _i32),
            pltpu.VMEM((C,), _i32),
            pltpu.VMEM((C,), _f32),
            pltpu.VMEM((C,), _f32),
            pltpu.VMEM((C,), _f32),
            pltpu.VMEM((2 * C, 8), _f32),
            pltpu.VMEM((2 * C, 8), _f32),
            pltpu.VMEM((C,), _f32),
            pltpu.VMEM((C,), _f32),
            pltpu.VMEM((C,), _f32),
            pltpu.SemaphoreType.DMA,
        ],
    )
    return grid_kernel(table, rays, maski)


def kernel(sky_cube_map, rays_d, mask):
    table = sky_cube_map.reshape(NWIN, 8)
    rays = rays_d.reshape(NPIX * 3)
    maski = mask.reshape(NPIX).astype(_i32)
    out = _sky(table, rays, maski)
    return out.reshape(3, H, W)


# speedup vs baseline: 2.6370x; 1.3937x over previous
import jax
import jax.numpy as jnp
from jax import lax
from jax.experimental import pallas as pl
from jax.experimental.pallas import tpu as pltpu
from jax.experimental.pallas import tpu_sc as plsc

RES = 1024
H, W = 1080, 1920
NPIX = H * W
NWIN = 6 * RES * RES * 3 // 8
KMAX = NWIN - 1
C = 1024
NCHUNK = NPIX // C
NLANE = 16
NVREG = C // NLANE
NC, NS = 2, 16
NW = NC * NS
TR, TCW = H // 8, W // 128

_f32 = jnp.float32
_i32 = jnp.int32


def _pix_coords(x, y, z):
    ax, ay, az = jnp.abs(x), jnp.abs(y), jnp.abs(z)
    is_x = (ax >= ay) & (ax >= az)
    is_y = jnp.logical_not(is_x) & (ay >= az)
    pos_x = x >= 0.0
    pos_y = y >= 0.0
    pos_z = z >= 0.0
    face = jnp.where(
        is_x, jnp.where(pos_x, 0, 1),
        jnp.where(is_y, jnp.where(pos_y, 2, 3),
                  jnp.where(pos_z, 4, 5))).astype(_i32)
    ma = jnp.where(is_x, ax, jnp.where(is_y, ay, az)) + 1e-30
    u = jnp.where(is_x, jnp.where(pos_x, -z, z),
                  jnp.where(is_y, x, jnp.where(pos_z, x, -x)))
    v = jnp.where(is_x, -y,
                  jnp.where(is_y, jnp.where(pos_y, z, -z), -y))
    inv = 1.0 / ma
    s = (u * inv) * (RES * 0.5) + (RES * 0.5 - 0.5)
    t = (v * inv) * (RES * 0.5) + (RES * 0.5 - 0.5)
    si = s.astype(_i32)
    ti = t.astype(_i32)
    s0 = si - jnp.where(s < si.astype(_f32), 1, 0).astype(_i32)
    t0 = ti - jnp.where(t < ti.astype(_f32), 1, 0).astype(_i32)
    ws = s - s0.astype(_f32)
    wt = t - t0.astype(_f32)
    s0c = jnp.clip(s0, 0, RES - 1)
    s1c = jnp.minimum(s0c + 1, RES - 1)
    t0c = jnp.clip(t0, 0, RES - 1)
    t1c = jnp.minimum(t0c + 1, RES - 1)
    sstep = s1c - s0c
    i00 = face * (RES * RES) + t0c * RES + s0c
    i10 = face * (RES * RES) + t1c * RES + s0c
    return i00, i10, sstep, ws, wt


def _body(table, rays, maski, out,
          x_v, y_v, z_v, mask_v, it0, it1, off0_v, off1_v, ds_v,
          ws_v, wt_v, mf_v, g0, g1, o0, o1, o2, sem):
    wid = lax.axis_index("s") * NC + lax.axis_index("c")
    cnt = (NCHUNK - wid + NW - 1) // NW
    lane = lax.iota(_i32, NLANE)

    def chunk_body(i, _):
        ch = wid + i * NW
        pbase = ch * C
        pltpu.sync_copy(rays.at[pl.ds(pbase, C)], x_v)
        pltpu.sync_copy(rays.at[pl.ds(NPIX + pbase, C)], y_v)
        pltpu.sync_copy(rays.at[pl.ds(2 * NPIX + pbase, C)], z_v)
        pltpu.sync_copy(maski.at[pl.ds(pbase, C)], mask_v)

        def pass1(j, _):
            d = pl.ds(j * NLANE, NLANE)
            x = x_v[d]
            y = y_v[d]
            z = z_v[d]
            i00, i10, sstep, ws, wt = _pix_coords(x, y, z)
            m = mask_v[d]
            w0 = i00 * 3
            w1 = i10 * 3
            k0 = lax.shift_right_logical(w0, 3)
            k1 = lax.shift_right_logical(w1, 3)
            k0b = jnp.minimum(k0 + 1, KMAX)
            k1b = jnp.minimum(k1 + 1, KMAX)
            row2 = (lane + j * NLANE) * 2
            plsc.store_scatter(it0, [row2], k0)
            plsc.store_scatter(it0, [row2 + 1], k0b)
            plsc.store_scatter(it1, [row2], k1)
            plsc.store_scatter(it1, [row2 + 1], k1b)
            off0_v[d] = w0 & 7
            off1_v[d] = w1 & 7
            ds_v[d] = sstep * 3
            ws_v[d] = ws
            wt_v[d] = wt
            mf_v[d] = m.astype(_f32)
            return _

        lax.fori_loop(0, NVREG, pass1, None)

        copies = []
        for idx_v, g in ((it0, g0), (it1, g1)):
            copies.append(pltpu.async_copy(table.at[idx_v], g, sem))
        for cp in copies:
            cp.wait()

        def pass2(j, _):
            d = pl.ds(j * NLANE, NLANE)
            row2 = (lane + j * NLANE) * 2
            ws = ws_v[d]
            wt = wt_v[d]
            mf = mf_v[d]
            off0 = off0_v[d]
            off1 = off1_v[d]
            ds = ds_v[d]
            for c, o_v in ((0, o0), (1, o1), (2, o2)):
                oa = off0 + c
                ob = oa + ds
                c00 = plsc.load_gather(
                    g0, [row2 + lax.shift_right_logical(oa, 3), oa & 7])
                c01 = plsc.load_gather(
                    g0, [row2 + lax.shift_right_logical(ob, 3), ob & 7])
                oa = off1 + c
                ob = oa + ds
                c10 = plsc.load_gather(
                    g1, [row2 + lax.shift_right_logical(oa, 3), oa & 7])
                c11 = plsc.load_gather(
                    g1, [row2 + lax.shift_right_logical(ob, 3), ob & 7])
                top = c00 + ws * (c01 - c00)
                bot = c10 + ws * (c11 - c10)
                val = top + wt * (bot - top)
                val = jnp.minimum(jnp.maximum(val, 0.0), 1.0) * mf
                o_v[d] = val
            return _

        lax.fori_loop(0, NVREG, pass2, None)

        pltpu.sync_copy(o0, out.at[pl.ds(pbase, C)])
        pltpu.sync_copy(o1, out.at[pl.ds(NPIX + pbase, C)])
        pltpu.sync_copy(o2, out.at[pl.ds(2 * NPIX + pbase, C)])
        return _

    lax.fori_loop(0, cnt, chunk_body, None)


@jax.jit
def _sky(table, rays, maski):
    grid_kernel = pl.kernel(
        _body,
        out_type=jax.ShapeDtypeStruct((3 * NPIX,), _f32),
        mesh=plsc.VectorSubcoreMesh(core_axis_name="c", subcore_axis_name="s"),
        compiler_params=pltpu.CompilerParams(
            needs_layout_passes=False, use_tc_tiling_on_sc=False),
        scratch_types=[
            pltpu.VMEM((C,), _f32),
            pltpu.VMEM((C,), _f32),
            pltpu.VMEM((C,), _f32),
            pltpu.VMEM((C,), _i32),
            pltpu.VMEM((2 * C,), _i32),
            pltpu.VMEM((2 * C,), _i32),
            pltpu.VMEM((C,), _i32),
            pltpu.VMEM((C,), _i32),
            pltpu.VMEM((C,), _i32),
            pltpu.VMEM((C,), _f32),
            pltpu.VMEM((C,), _f32),
            pltpu.VMEM((C,), _f32),
            pltpu.VMEM((2 * C, 8), _f32),
            pltpu.VMEM((2 * C, 8), _f32),
            pltpu.VMEM((C,), _f32),
            pltpu.VMEM((C,), _f32),
            pltpu.VMEM((C,), _f32),
            pltpu.SemaphoreType.DMA,
        ],
    )
    return grid_kernel(table, rays, maski)


def _to_tile_order(x):
    return (x.reshape(TR, 8, TCW, 128)
            .transpose(0, 2, 1, 3)
            .reshape(NPIX))


def _from_tile_order(x):
    return (x.reshape(TR, TCW, 8, 128)
            .transpose(0, 2, 1, 3)
            .reshape(H, W))


def kernel(sky_cube_map, rays_d, mask):
    table = jnp.maximum(sky_cube_map, 0.0).reshape(NWIN, 8)
    rays_pl = jnp.transpose(rays_d, (2, 0, 1))
    rays = jnp.stack([_to_tile_order(rays_pl[c]) for c in range(3)]
                     ).reshape(3 * NPIX)
    maski = _to_tile_order(mask.astype(_i32))
    out = _sky(table, rays, maski)
    planes = [_from_tile_order(out[c * NPIX:(c + 1) * NPIX]) for c in range(3)]
    return jnp.stack(planes)


# speedup vs baseline: 27.6088x; 10.4697x over previous
import jax
import jax.numpy as jnp
from jax import lax
from jax.experimental import pallas as pl
from jax.experimental.pallas import tpu as pltpu
from jax.experimental.pallas import tpu_sc as plsc

RES = 1024
H, W = 1080, 1920
NPIX = H * W
NTEX = 6 * RES * RES
PLANE = RES * RES
NWIN = NTEX * 4 // 8
C = 1024
NCHUNK = NPIX // C
CA = 2048
NCHA = NTEX // CA
CHF = PLANE // CA
NLANE = 16
NVREG = C // NLANE
NC, NS = 2, 16
NW = NC * NS
TR, TCW = H // 8, W // 128

_f32 = jnp.float32
_i32 = jnp.int32


def _pix_coords(x, y, z):
    ax, ay, az = jnp.abs(x), jnp.abs(y), jnp.abs(z)
    is_x = (ax >= ay) & (ax >= az)
    is_y = jnp.logical_not(is_x) & (ay >= az)
    pos_x = x >= 0.0
    pos_y = y >= 0.0
    pos_z = z >= 0.0
    face = jnp.where(
        is_x, jnp.where(pos_x, 0, 1),
        jnp.where(is_y, jnp.where(pos_y, 2, 3),
                  jnp.where(pos_z, 4, 5))).astype(_i32)
    ma = jnp.where(is_x, ax, jnp.where(is_y, ay, az)) + 1e-30
    u = jnp.where(is_x, jnp.where(pos_x, -z, z),
                  jnp.where(is_y, x, jnp.where(pos_z, x, -x)))
    v = jnp.where(is_x, -y,
                  jnp.where(is_y, jnp.where(pos_y, z, -z), -y))
    inv = 1.0 / ma
    s = (u * inv) * (RES * 0.5) + (RES * 0.5 - 0.5)
    t = (v * inv) * (RES * 0.5) + (RES * 0.5 - 0.5)
    si = s.astype(_i32)
    ti = t.astype(_i32)
    s0 = si - jnp.where(s < si.astype(_f32), 1, 0).astype(_i32)
    t0 = ti - jnp.where(t < ti.astype(_f32), 1, 0).astype(_i32)
    ws = s - s0.astype(_f32)
    wt = t - t0.astype(_f32)
    s0c = jnp.clip(s0, 0, RES - 1)
    s1c = jnp.minimum(s0c + 1, RES - 1)
    t0c = jnp.clip(t0, 0, RES - 1)
    t1c = jnp.minimum(t0c + 1, RES - 1)
    return face, s0c, s1c, t0c, t1c, ws, wt


def _ibody(planes, out, r_v, g_v, b_v, o_v, sem):
    wid = lax.axis_index("s") * NC + lax.axis_index("c")
    cnt = (NCHA - wid + NW - 1) // NW
    lane = lax.iota(_i32, NLANE)

    def chunk_body(i, _):
        ch = wid + i * NW
        f = ch // CHF
        base = f * (3 * PLANE) + (ch % CHF) * CA
        pltpu.sync_copy(planes.at[pl.ds(base, CA)], r_v)
        pltpu.sync_copy(planes.at[pl.ds(base + PLANE, CA)], g_v)
        pltpu.sync_copy(planes.at[pl.ds(base + 2 * PLANE, CA)], b_v)

        def vl(j, _):
            d = pl.ds(j * NLANE, NLANE)
            idx4 = (lane + j * NLANE) * 4
            plsc.store_scatter(o_v, [idx4], r_v[d])
            plsc.store_scatter(o_v, [idx4 + 1], g_v[d])
            plsc.store_scatter(o_v, [idx4 + 2], b_v[d])
            return _

        lax.fori_loop(0, CA // NLANE, vl, None)
        pltpu.sync_copy(o_v, out.at[pl.ds(ch * (CA * 4), CA * 4)])
        return _

    lax.fori_loop(0, cnt, chunk_body, None)


def _body(table, rays, maski, out,
          x_v, y_v, z_v, mask_v, it0, it1, offa_v, offb_v,
          ws_v, wt_v, mf_v, g0, g1, o0, o1, o2, sem):
    wid = lax.axis_index("s") * NC + lax.axis_index("c")
    cnt = (NCHUNK - wid + NW - 1) // NW
    lane = lax.iota(_i32, NLANE)

    def chunk_body(i, _):
        ch = wid + i * NW
        pbase = ch * C
        pltpu.sync_copy(rays.at[pl.ds(pbase, C)], x_v)
        pltpu.sync_copy(rays.at[pl.ds(NPIX + pbase, C)], y_v)
        pltpu.sync_copy(rays.at[pl.ds(2 * NPIX + pbase, C)], z_v)
        pltpu.sync_copy(maski.at[pl.ds(pbase, C)], mask_v)

        def pass1(j, _):
            d = pl.ds(j * NLANE, NLANE)
            face, s0c, s1c, t0c, t1c, ws, wt = _pix_coords(
                x_v[d], y_v[d], z_v[d])
            m = mask_v[d]
            sp0 = ((s0c >> 7) << 10) + (s0c & 127)
            sp1 = ((s1c >> 7) << 10) + (s1c & 127)
            base0 = face * PLANE + ((t0c >> 3) << 13) + ((t0c & 7) << 7)
            base1 = face * PLANE + ((t1c >> 3) << 13) + ((t1c & 7) << 7)
            row2 = (lane + j * NLANE) * 2
            plsc.store_scatter(it0, [row2], (base0 + sp0) >> 1)
            plsc.store_scatter(it0, [row2 + 1], (base0 + sp1) >> 1)
            plsc.store_scatter(it1, [row2], (base1 + sp0) >> 1)
            plsc.store_scatter(it1, [row2 + 1], (base1 + sp1) >> 1)
            offa_v[d] = (s0c & 1) << 2
            offb_v[d] = (s1c & 1) << 2
            ws_v[d] = ws
            wt_v[d] = wt
            mf_v[d] = m.astype(_f32)
            return _

        lax.fori_loop(0, NVREG, pass1, None)

        copies = []
        for idx_v, g in ((it0, g0), (it1, g1)):
            copies.append(pltpu.async_copy(table.at[idx_v], g, sem))
        for cp in copies:
            cp.wait()

        def pass2(j, _):
            d = pl.ds(j * NLANE, NLANE)
            row2 = (lane + j * NLANE) * 2
            ws = ws_v[d]
            wt = wt_v[d]
            mf = mf_v[d]
            offa = offa_v[d]
            offb = offb_v[d]
            for c, o_v in ((0, o0), (1, o1), (2, o2)):
                ca = offa + c
                cb = offb + c
                c00 = plsc.load_gather(g0, [row2, ca])
                c01 = plsc.load_gather(g0, [row2 + 1, cb])
                c10 = plsc.load_gather(g1, [row2, ca])
                c11 = plsc.load_gather(g1, [row2 + 1, cb])
                top = c00 + ws * (c01 - c00)
                bot = c10 + ws * (c11 - c10)
                val = top + wt * (bot - top)
                val = jnp.minimum(jnp.maximum(val, 0.0), 1.0) * mf
                o_v[d] = val
            return _

        lax.fori_loop(0, NVREG, pass2, None)

        pltpu.sync_copy(o0, out.at[pl.ds(pbase, C)])
        pltpu.sync_copy(o1, out.at[pl.ds(NPIX + pbase, C)])
        pltpu.sync_copy(o2, out.at[pl.ds(2 * NPIX + pbase, C)])
        return _

    lax.fori_loop(0, cnt, chunk_body, None)


_SC_PARAMS = pltpu.CompilerParams(
    needs_layout_passes=False, use_tc_tiling_on_sc=False)
_MESH = plsc.VectorSubcoreMesh(core_axis_name="c", subcore_axis_name="s")


@jax.jit
def _run(planes, rays, maski):
    interleave = pl.kernel(
        _ibody,
        out_type=jax.ShapeDtypeStruct((NTEX * 4,), _f32),
        mesh=_MESH,
        compiler_params=_SC_PARAMS,
        scratch_types=[
            pltpu.VMEM((CA,), _f32),
            pltpu.VMEM((CA,), _f32),
            pltpu.VMEM((CA,), _f32),
            pltpu.VMEM((CA * 4,), _f32),
            pltpu.SemaphoreType.DMA,
        ],
    )
    rgba = interleave(planes)
    table = rgba.reshape(NWIN, 8)
    lookup = pl.kernel(
        _body,
        out_type=jax.ShapeDtypeStruct((3 * NPIX,), _f32),
        mesh=_MESH,
        compiler_params=_SC_PARAMS,
        scratch_types=[
            pltpu.VMEM((C,), _f32),
            pltpu.VMEM((C,), _f32),
            pltpu.VMEM((C,), _f32),
            pltpu.VMEM((C,), _i32),
            pltpu.VMEM((2 * C,), _i32),
            pltpu.VMEM((2 * C,), _i32),
            pltpu.VMEM((C,), _i32),
            pltpu.VMEM((C,), _i32),
            pltpu.VMEM((C,), _f32),
            pltpu.VMEM((C,), _f32),
            pltpu.VMEM((C,), _f32),
            pltpu.VMEM((2 * C, 8), _f32),
            pltpu.VMEM((2 * C, 8), _f32),
            pltpu.VMEM((C,), _f32),
            pltpu.VMEM((C,), _f32),
            pltpu.VMEM((C,), _f32),
            pltpu.SemaphoreType.DMA,
        ],
    )
    return lookup(table, rays, maski)


def _to_tile_order(x):
    return (x.reshape(TR, 8, TCW, 128)
            .transpose(0, 2, 1, 3)
            .reshape(NPIX))


def _from_tile_order(x):
    return (x.reshape(TR, TCW, 8, 128)
            .transpose(0, 2, 1, 3)
            .reshape(H, W))


def kernel(sky_cube_map, rays_d, mask):
    planes = (jnp.transpose(sky_cube_map, (0, 3, 1, 2))
              .reshape(6, 3, RES // 8, 8, RES // 128, 128)
              .transpose(0, 1, 2, 4, 3, 5)
              .reshape(NTEX * 3))
    rays_pl = jnp.transpose(rays_d, (2, 0, 1))
    rays = jnp.stack([_to_tile_order(rays_pl[c]) for c in range(3)]
                     ).reshape(3 * NPIX)
    maski = _to_tile_order(mask.astype(_i32))
    out = _run(planes, rays, maski)
    planes_o = [_from_tile_order(out[c * NPIX:(c + 1) * NPIX])
                for c in range(3)]
    return jnp.stack(planes_o)
